# Initial kernel scaffold; baseline (speedup 1.0000x reference)
import jax
import jax.numpy as jnp
from jax.experimental import pallas as pl


def kernel(x, circle_index, W0, b0, W1, b1, W2, b2, Wf, bf):
    raise NotImplementedError("write your pallas kernel here")


# speedup vs baseline: 3.0799x; 3.0799x over previous
import functools

import jax
import jax.numpy as jnp
from jax import lax
from jax.experimental import pallas as pl
from jax.experimental.pallas import tpu as pltpu
from jax.experimental.pallas import tpu_sc as plsc

H = 256
CS = 18
OUT_C = 16

NP = 51200
ROWS = NP * CS
NW = 32
B_PER_W = ROWS // NW
CHUNK = 128
N_CHUNKS = B_PER_W // CHUNK

NODE_BLK = 64


def _sc_gather_body(x_hbm, idx_hbm, out_hbm, idx_v, rows_v, sem):
    wid = lax.axis_index("s") * 2 + lax.axis_index("c")
    base = wid * B_PER_W

    def body(k, carry):
        off = base + k * CHUNK
        pltpu.sync_copy(idx_hbm.at[pl.ds(off, CHUNK)], idx_v)
        pltpu.async_copy(x_hbm.at[idx_v], rows_v, sem).wait()
        pltpu.sync_copy(rows_v, out_hbm.at[pl.ds(off, CHUNK)])
        return carry

    lax.fori_loop(0, N_CHUNKS, body, 0)


_sc_gather = functools.partial(
    pl.kernel,
    mesh=plsc.VectorSubcoreMesh(core_axis_name="c", subcore_axis_name="s"),
    out_type=jax.ShapeDtypeStruct((ROWS, H), jnp.float32),
    scratch_types=[
        pltpu.VMEM((CHUNK,), jnp.int32),
        pltpu.VMEM((CHUNK, H), jnp.float32),
        pltpu.SemaphoreType.DMA,
    ],
)(_sc_gather_body)


def _tc_block(xg_ref, ws_ref, bs_ref, wf_ref, bf_ref, out_ref):
    xb = xg_ref[...]
    y0 = xb[:, 0:OUT_C, :].reshape(NODE_BLK * OUT_C, H)
    y1 = xb[:, 1:OUT_C + 1, :].reshape(NODE_BLK * OUT_C, H)
    y2 = xb[:, 2:OUT_C + 2, :].reshape(NODE_BLK * OUT_C, H)
    y = jnp.concatenate([y0, y1, y2], axis=1)
    s = jnp.dot(y, ws_ref[...], preferred_element_type=jnp.float32)
    s = s + bs_ref[...]
    e = jnp.where(s > 0, s, jnp.exp(jnp.minimum(s, 0.0)) - 1.0)
    o = jnp.dot(e, wf_ref[...], preferred_element_type=jnp.float32)
    o = o + bf_ref[...]
    out_ref[...] = o.reshape(NODE_BLK, OUT_C, H)


def kernel(x, circle_index, W0, b0, W1, b1, W2, b2, Wf, bf):
    n_nodes = x.shape[0]
    ci = circle_index.astype(jnp.int32)
    flat = ci.reshape(-1)
    idx = jnp.concatenate(
        [flat, jnp.zeros((ROWS - flat.shape[0],), jnp.int32)])

    xg = _sc_gather(x, idx)
    xg3 = xg.reshape(NP, CS, H)

    ws = jnp.concatenate([W0.T, W1.T, W2.T], axis=0)
    bs = (b0 + b1 + b2).reshape(1, H)
    wf_t = Wf.T
    bf2 = bf.reshape(1, H)

    grid = (n_nodes + NODE_BLK - 1) // NODE_BLK
    out = pl.pallas_call(
        _tc_block,
        grid=(grid,),
        in_specs=[
            pl.BlockSpec((NODE_BLK, CS, H), lambda i: (i, 0, 0)),
            pl.BlockSpec((3 * H, H), lambda i: (0, 0)),
            pl.BlockSpec((1, H), lambda i: (0, 0)),
            pl.BlockSpec((H, H), lambda i: (0, 0)),
            pl.BlockSpec((1, H), lambda i: (0, 0)),
        ],
        out_specs=pl.BlockSpec((NODE_BLK, OUT_C, H), lambda i: (i, 0, 0)),
        out_shape=jax.ShapeDtypeStruct((n_nodes, OUT_C, H), jnp.float32),
    )(xg3, ws, bs, wf_t, bf2)

    padding_size = jnp.sum(ci == -1, axis=-1)
    num_neighbors = jnp.where(padding_size == CS, 0, CS - padding_size - 2)
    msg_mask = jnp.arange(1, OUT_C + 1)[None, :] <= num_neighbors[:, None]
    return out, msg_mask


# speedup vs baseline: 3.1782x; 1.0319x over previous
import functools

import jax
import jax.numpy as jnp
from jax import lax
from jax.experimental import pallas as pl
from jax.experimental.pallas import tpu as pltpu
from jax.experimental.pallas import tpu_sc as plsc

H = 256
CS = 18
OUT_C = 16

NP = 51200
ROWS = NP * CS
NW = 32
B_PER_W = ROWS // NW
CHUNK = 128
N_CHUNKS = B_PER_W // CHUNK

NODE_BLK = 64


def _sc_gather_body(x_hbm, idx_hbm, out_hbm, idx_v, buf0, buf1,
                    sem_g0, sem_g1, sem_w0, sem_w1):
    wid = lax.axis_index("s") * 2 + lax.axis_index("c")
    base = wid * B_PER_W

    def gather(k, buf, sem):
        pltpu.async_copy(x_hbm.at[idx_v.at[k]], buf, sem)

    def write(k, buf, sem):
        pltpu.async_copy(buf, out_hbm.at[pl.ds(base + k * CHUNK, CHUNK)], sem)

    pltpu.sync_copy(idx_hbm.at[wid], idx_v)
    gather(0, buf0, sem_g0)
    pltpu.make_async_copy(x_hbm.at[idx_v.at[0]], buf0, sem_g0).wait()
    write(0, buf0, sem_w0)
    gather(1, buf1, sem_g1)

    def wait_g(buf, sem, k):
        pltpu.make_async_copy(x_hbm.at[idx_v.at[k]], buf, sem).wait()

    def wait_w(buf, sem, k):
        pltpu.make_async_copy(
            buf, out_hbm.at[pl.ds(base + k * CHUNK, CHUNK)], sem).wait()

    def body(j, carry):
        a = 2 * j + 1
        wait_g(buf1, sem_g1, a)
        write(a, buf1, sem_w1)
        wait_w(buf0, sem_w0, a - 1)
        gather(a + 1, buf0, sem_g0)
        wait_w(buf1, sem_w1, a)
        gather(a + 2, buf1, sem_g1)
        wait_g(buf0, sem_g0, a + 1)
        write(a + 1, buf0, sem_w0)
        return carry

    lax.fori_loop(0, (N_CHUNKS - 1) // 2, body, 0)
    wait_g(buf1, sem_g1, N_CHUNKS)
    wait_w(buf0, sem_w0, N_CHUNKS - 1)


_sc_gather = functools.partial(
    pl.kernel,
    mesh=plsc.VectorSubcoreMesh(core_axis_name="c", subcore_axis_name="s"),
    out_type=jax.ShapeDtypeStruct((ROWS, H), jnp.float32),
    scratch_types=[
        pltpu.VMEM((N_CHUNKS + 1, CHUNK), jnp.int32),
        pltpu.VMEM((CHUNK, H), jnp.float32),
        pltpu.VMEM((CHUNK, H), jnp.float32),
        pltpu.SemaphoreType.DMA,
        pltpu.SemaphoreType.DMA,
        pltpu.SemaphoreType.DMA,
        pltpu.SemaphoreType.DMA,
    ],
)(_sc_gather_body)


def _tc_block(xg_ref, ws_ref, bs_ref, wf_ref, bf_ref, out_ref):
    xb = xg_ref[...]
    y0 = xb[:, 0:OUT_C, :].reshape(NODE_BLK * OUT_C, H)
    y1 = xb[:, 1:OUT_C + 1, :].reshape(NODE_BLK * OUT_C, H)
    y2 = xb[:, 2:OUT_C + 2, :].reshape(NODE_BLK * OUT_C, H)
    y = jnp.concatenate([y0, y1, y2], axis=1)
    s = jnp.dot(y, ws_ref[...], preferred_element_type=jnp.float32)
    s = s + bs_ref[...]
    e = jnp.where(s > 0, s, jnp.exp(jnp.minimum(s, 0.0)) - 1.0)
    o = jnp.dot(e, wf_ref[...], preferred_element_type=jnp.float32)
    o = o + bf_ref[...]
    out_ref[...] = o.reshape(NODE_BLK, OUT_C, H)


def kernel(x, circle_index, W0, b0, W1, b1, W2, b2, Wf, bf):
    n_nodes = x.shape[0]
    ci = circle_index.astype(jnp.int32)
    flat = ci.reshape(-1)
    idx = jnp.concatenate(
        [flat, jnp.zeros((ROWS - flat.shape[0],), jnp.int32)])
    idx = jnp.pad(idx.reshape(NW, N_CHUNKS, CHUNK), ((0, 0), (0, 1), (0, 0)))

    xg = _sc_gather(x, idx)
    xg3 = xg.reshape(NP, CS, H)

    ws = jnp.concatenate([W0.T, W1.T, W2.T], axis=0)
    bs = (b0 + b1 + b2).reshape(1, H)
    wf_t = Wf.T
    bf2 = bf.reshape(1, H)

    grid = (n_nodes + NODE_BLK - 1) // NODE_BLK
    out = pl.pallas_call(
        _tc_block,
        grid=(grid,),
        in_specs=[
            pl.BlockSpec((NODE_BLK, CS, H), lambda i: (i, 0, 0)),
            pl.BlockSpec((3 * H, H), lambda i: (0, 0)),
            pl.BlockSpec((1, H), lambda i: (0, 0)),
            pl.BlockSpec((H, H), lambda i: (0, 0)),
            pl.BlockSpec((1, H), lambda i: (0, 0)),
        ],
        out_specs=pl.BlockSpec((NODE_BLK, OUT_C, H), lambda i: (i, 0, 0)),
        out_shape=jax.ShapeDtypeStruct((n_nodes, OUT_C, H), jnp.float32),
    )(xg3, ws, bs, wf_t, bf2)

    padding_size = jnp.sum(ci == -1, axis=-1)
    num_neighbors = jnp.where(padding_size == CS, 0, CS - padding_size - 2)
    msg_mask = jnp.arange(1, OUT_C + 1)[None, :] <= num_neighbors[:, None]
    return out, msg_mask


# speedup vs baseline: 4.1433x; 1.3037x over previous
import functools

import jax
import jax.numpy as jnp
from jax import lax
from jax.experimental import pallas as pl
from jax.experimental.pallas import tpu as pltpu
from jax.experimental.pallas import tpu_sc as plsc

H = 256
CS = 18
OUT_C = 16

NP = 51200
ROWS = NP * CS
NW = 32
B_PER_W = ROWS // NW
CHUNK = 128
N_CHUNKS = B_PER_W // CHUNK

NODE_BLK = 400


def _sc_gather_body(x_hbm, idx_hbm, out_hbm, idx_v, buf0, buf1,
                    sem_g0, sem_g1, sem_w0, sem_w1):
    wid = lax.axis_index("s") * 2 + lax.axis_index("c")
    base = wid * B_PER_W

    def gather(k, buf, sem):
        pltpu.async_copy(x_hbm.at[idx_v.at[k]], buf, sem)

    def write(k, buf, sem):
        pltpu.async_copy(buf, out_hbm.at[pl.ds(base + k * CHUNK, CHUNK)], sem)

    pltpu.sync_copy(idx_hbm.at[wid], idx_v)
    gather(0, buf0, sem_g0)
    pltpu.make_async_copy(x_hbm.at[idx_v.at[0]], buf0, sem_g0).wait()
    write(0, buf0, sem_w0)
    gather(1, buf1, sem_g1)

    def wait_g(buf, sem, k):
        pltpu.make_async_copy(x_hbm.at[idx_v.at[k]], buf, sem).wait()

    def wait_w(buf, sem, k):
        pltpu.make_async_copy(
            buf, out_hbm.at[pl.ds(base + k * CHUNK, CHUNK)], sem).wait()

    def body(j, carry):
        a = 2 * j + 1
        wait_g(buf1, sem_g1, a)
        write(a, buf1, sem_w1)
        wait_w(buf0, sem_w0, a - 1)
        gather(a + 1, buf0, sem_g0)
        wait_w(buf1, sem_w1, a)
        gather(a + 2, buf1, sem_g1)
        wait_g(buf0, sem_g0, a + 1)
        write(a + 1, buf0, sem_w0)
        return carry

    lax.fori_loop(0, (N_CHUNKS - 1) // 2, body, 0)
    wait_g(buf1, sem_g1, N_CHUNKS)
    wait_w(buf0, sem_w0, N_CHUNKS - 1)


_sc_gather = functools.partial(
    pl.kernel,
    mesh=plsc.VectorSubcoreMesh(core_axis_name="c", subcore_axis_name="s"),
    out_type=jax.ShapeDtypeStruct((ROWS, H), jnp.float32),
    scratch_types=[
        pltpu.VMEM((N_CHUNKS + 1, CHUNK), jnp.int32),
        pltpu.VMEM((CHUNK, H), jnp.float32),
        pltpu.VMEM((CHUNK, H), jnp.float32),
        pltpu.SemaphoreType.DMA,
        pltpu.SemaphoreType.DMA,
        pltpu.SemaphoreType.DMA,
        pltpu.SemaphoreType.DMA,
    ],
)(_sc_gather_body)


def _tc_block(xg_ref, ws_ref, bs_ref, wf_ref, bf_ref, out_ref):
    for c in range(OUT_C):
        y = jnp.concatenate(
            [xg_ref[c], xg_ref[c + 1], xg_ref[c + 2]], axis=1)
        s = jnp.dot(y, ws_ref[...], preferred_element_type=jnp.float32)
        s = s + bs_ref[...]
        e = jnp.where(s > 0, s, jnp.exp(jnp.minimum(s, 0.0)) - 1.0)
        o = jnp.dot(e, wf_ref[...], preferred_element_type=jnp.float32)
        out_ref[:, c, :] = o + bf_ref[...]


def kernel(x, circle_index, W0, b0, W1, b1, W2, b2, Wf, bf):
    n_nodes = x.shape[0]
    ci = circle_index.astype(jnp.int32)
    cit = jnp.pad(ci.T, ((0, 0), (0, NP - n_nodes)))
    idx = cit.reshape(-1)
    idx = jnp.pad(idx.reshape(NW, N_CHUNKS, CHUNK), ((0, 0), (0, 1), (0, 0)))

    xg = _sc_gather(x, idx)
    xg3 = xg.reshape(CS, NP, H)

    ws = jnp.concatenate([W0.T, W1.T, W2.T], axis=0)
    bs = (b0 + b1 + b2).reshape(1, H)
    wf_t = Wf.T
    bf2 = bf.reshape(1, H)

    grid = n_nodes // NODE_BLK
    out = pl.pallas_call(
        _tc_block,
        grid=(grid,),
        in_specs=[
            pl.BlockSpec((CS, NODE_BLK, H), lambda i: (0, i, 0)),
            pl.BlockSpec((3 * H, H), lambda i: (0, 0)),
            pl.BlockSpec((1, H), lambda i: (0, 0)),
            pl.BlockSpec((H, H), lambda i: (0, 0)),
            pl.BlockSpec((1, H), lambda i: (0, 0)),
        ],
        out_specs=pl.BlockSpec((NODE_BLK, OUT_C, H), lambda i: (i, 0, 0)),
        out_shape=jax.ShapeDtypeStruct((n_nodes, OUT_C, H), jnp.float32),
    )(xg3, ws, bs, wf_t, bf2)

    padding_size = jnp.sum(ci == -1, axis=-1)
    num_neighbors = jnp.where(padding_size == CS, 0, CS - padding_size - 2)
    msg_mask = jnp.arange(1, OUT_C + 1)[None, :] <= num_neighbors[:, None]
    return out, msg_mask


# speedup vs baseline: 4.1933x; 1.0121x over previous
import functools

import jax
import jax.numpy as jnp
from jax import lax
from jax.experimental import pallas as pl
from jax.experimental.pallas import tpu as pltpu
from jax.experimental.pallas import tpu_sc as plsc

H = 256
CS = 18
OUT_C = 16

NP = 51200
ROWS = NP * CS
NW = 32
CHUNK = 128
N_CHUNKS_TOT = ROWS // CHUNK
PAIR_CHUNKS = N_CHUNKS_TOT // 16
CORE0_CHUNKS = 129
CORE1_CHUNKS = PAIR_CHUNKS - CORE0_CHUNKS
MAX_CHUNKS = max(CORE0_CHUNKS, CORE1_CHUNKS)
SLAB = ((MAX_CHUNKS + 1 + 7) // 8 + 1) * 8
IDX_ROWS = ((N_CHUNKS_TOT + SLAB + 7) // 8) * 8

NODE_BLK = 400


def _sc_gather_body(x_hbm, idx_hbm, out_hbm, idx_v, buf0, buf1,
                    sem_g0, sem_g1, sem_w0, sem_w1):
    c = lax.axis_index("c")
    s = lax.axis_index("s")
    start = s * PAIR_CHUNKS + jnp.where(c == 0, 0, CORE0_CHUNKS)
    nloc = jnp.where(c == 0, CORE0_CHUNKS, CORE1_CHUNKS)
    astart = pl.multiple_of((start // 8) * 8, 8)
    off = start - astart

    def gather(k, buf, sem):
        pltpu.async_copy(x_hbm.at[idx_v.at[off + k]], buf, sem)

    def write(k, buf, sem):
        pltpu.async_copy(
            buf, out_hbm.at[pl.ds((start + k) * CHUNK, CHUNK)], sem)

    def wait_g(buf, sem, k):
        pltpu.make_async_copy(x_hbm.at[idx_v.at[off + k]], buf, sem).wait()

    def wait_w(buf, sem, k):
        pltpu.make_async_copy(
            buf, out_hbm.at[pl.ds((start + k) * CHUNK, CHUNK)], sem).wait()

    pltpu.sync_copy(idx_hbm.at[pl.ds(astart, SLAB)], idx_v)
    gather(0, buf0, sem_g0)
    wait_g(buf0, sem_g0, 0)
    write(0, buf0, sem_w0)
    gather(1, buf1, sem_g1)

    def body(j, carry):
        a = 2 * j + 1
        wait_g(buf1, sem_g1, a)
        write(a, buf1, sem_w1)
        wait_w(buf0, sem_w0, a - 1)
        gather(a + 1, buf0, sem_g0)
        wait_w(buf1, sem_w1, a)
        gather(a + 2, buf1, sem_g1)
        wait_g(buf0, sem_g0, a + 1)
        write(a + 1, buf0, sem_w0)
        return carry

    lax.fori_loop(0, (nloc - 1) // 2, body, 0)
    wait_g(buf1, sem_g1, nloc)
    wait_w(buf0, sem_w0, nloc - 1)


_sc_gather = functools.partial(
    pl.kernel,
    mesh=plsc.VectorSubcoreMesh(core_axis_name="c", subcore_axis_name="s"),
    out_type=jax.ShapeDtypeStruct((ROWS, H), jnp.float32),
    scratch_types=[
        pltpu.VMEM((SLAB, CHUNK), jnp.int32),
        pltpu.VMEM((CHUNK, H), jnp.float32),
        pltpu.VMEM((CHUNK, H), jnp.float32),
        pltpu.SemaphoreType.DMA,
        pltpu.SemaphoreType.DMA,
        pltpu.SemaphoreType.DMA,
        pltpu.SemaphoreType.DMA,
    ],
)(_sc_gather_body)


def _tc_block(xg_ref, ws_ref, bs_ref, wf_ref, bf_ref, out_ref):
    for c in range(OUT_C):
        y = jnp.concatenate(
            [xg_ref[c], xg_ref[c + 1], xg_ref[c + 2]], axis=1)
        s = jnp.dot(y, ws_ref[...], preferred_element_type=jnp.float32)
        s = s + bs_ref[...]
        e = jnp.where(s > 0, s, jnp.exp(jnp.minimum(s, 0.0)) - 1.0)
        o = jnp.dot(e, wf_ref[...], preferred_element_type=jnp.float32)
        out_ref[:, c, :] = o + bf_ref[...]


def kernel(x, circle_index, W0, b0, W1, b1, W2, b2, Wf, bf):
    n_nodes = x.shape[0]
    ci = circle_index.astype(jnp.int32)
    cit = jnp.pad(ci.T, ((0, 0), (0, NP - n_nodes)))
    idx = jnp.pad(cit.reshape(N_CHUNKS_TOT, CHUNK),
                  ((0, IDX_ROWS - N_CHUNKS_TOT), (0, 0)))

    xg = _sc_gather(x, idx)
    xg3 = xg.reshape(CS, NP, H)

    ws = jnp.concatenate([W0.T, W1.T, W2.T], axis=0)
    bs = (b0 + b1 + b2).reshape(1, H)
    wf_t = Wf.T
    bf2 = bf.reshape(1, H)

    grid = n_nodes // NODE_BLK
    out = pl.pallas_call(
        _tc_block,
        grid=(grid,),
        in_specs=[
            pl.BlockSpec((CS, NODE_BLK, H), lambda i: (0, i, 0)),
            pl.BlockSpec((3 * H, H), lambda i: (0, 0)),
            pl.BlockSpec((1, H), lambda i: (0, 0)),
            pl.BlockSpec((H, H), lambda i: (0, 0)),
            pl.BlockSpec((1, H), lambda i: (0, 0)),
        ],
        out_specs=pl.BlockSpec((NODE_BLK, OUT_C, H), lambda i: (i, 0, 0)),
        out_shape=jax.ShapeDtypeStruct((n_nodes, OUT_C, H), jnp.float32),
    )(xg3, ws, bs, wf_t, bf2)

    padding_size = jnp.sum(ci == -1, axis=-1)
    num_neighbors = jnp.where(padding_size == CS, 0, CS - padding_size - 2)
    msg_mask = jnp.arange(1, OUT_C + 1)[None, :] <= num_neighbors[:, None]
    return out, msg_mask
